# baseline (speedup 1.0000x reference)
import functools

import jax
import jax.numpy as jnp
import numpy as np
from jax import lax
from jax.experimental import pallas as pl
from jax.experimental.pallas import tpu as pltpu
from jax.experimental.pallas import tpu_sc as plsc

N = 10000
E = 160000
FC = 128
NG = 64
L = 2

EBLK = 4000


def _centers_step(vmin, vmax):
    step = np.float32((vmax - vmin) / (FC - 1))
    return step


def _rbf(x, vmin, vmax):
    step = _centers_step(vmin, vmax)
    gamma = np.float32(1.0) / (step * step)
    c = vmin + step * lax.broadcasted_iota(jnp.int32, (1, FC), 1).astype(
        jnp.float32)
    d = x - c
    return jnp.exp(-gamma * d * d)


def _softplus(x):
    return jnp.log1p(jnp.exp(x))


def _dot(a, b):
    return jnp.dot(a, b, preferred_element_type=jnp.float32)


def _edge_kernel(ea_ref, el_ref, an_ref, wfeat_ref, bfeat_ref, wqkv_ref,
                 wkvla_ref, wee_ref, ee1_ref, ee2_ref):
    ea = ea_ref[...]
    d = jnp.sqrt(jnp.sum(ea * ea, axis=1, keepdims=True))
    x_ef = jnp.float32(-0.75) / d
    f_ef = _rbf(x_ef, -4.0, 4.0)
    ef = _softplus(_dot(f_ef, wfeat_ref[:FC, :FC]) + bfeat_ref[:, :FC])

    nla = []
    for n in range(3):
        f_l = _rbf(el_ref[:, n:n + 1], -4.0, 4.0)
        f_a = _rbf(an_ref[:, n:n + 1], -1.0, 1.0)
        f_cat = jnp.concatenate([f_l, f_a], axis=1)
        nla.append(_softplus(_dot(f_cat, wfeat_ref[...]) + bfeat_ref[...]))

    for i in range(L):
        qkv = _dot(ef, wqkv_ref[i])
        q = qkv[:, :FC]
        kb = qkv[:, FC:2 * FC]
        vb = qkv[:, 2 * FC:]
        s = []
        v = []
        for n in range(3):
            kv = _dot(nla[n], wkvla_ref[i])
            s.append(jnp.sum(q * (kb + kv[:, :FC]), axis=1, keepdims=True))
            v.append(vb + kv[:, FC:])
        s0, s1, s2 = s
        e0 = jnp.exp(s0)
        e1 = jnp.exp(s1)
        e2 = jnp.exp(s2)
        inv_den = 1.0 / (e0 + e1 + e2)
        w0 = e0 * inv_den
        w1 = e1 * inv_den
        w2 = e2 * inv_den
        ef = ef + (w0 * v[0] + w1 * v[1] + w2 * v[2])
        out_ref = ee1_ref if i == 0 else ee2_ref
        out_ref[...] = _dot(ef, wee_ref[i])


def _edge_embeddings(edge_attr, edge_nei_len, edge_nei_angle, wfeat, bfeat,
                     wqkv, wkvla, wee):
    grid = (E // EBLK,)
    eb = pl.BlockSpec((EBLK, 3), lambda e: (e, 0))
    wspec = lambda shape: pl.BlockSpec(shape, lambda e: (0,) * len(shape))
    out = pl.pallas_call(
        _edge_kernel,
        grid=grid,
        in_specs=[
            eb, eb, eb,
            wspec((2 * FC, 2 * FC)),
            wspec((1, 2 * FC)),
            wspec((L, FC, 3 * FC)),
            wspec((L, 2 * FC, 2 * FC)),
            wspec((L, FC, FC)),
        ],
        out_specs=[
            pl.BlockSpec((EBLK, FC), lambda e: (e, 0)),
            pl.BlockSpec((EBLK, FC), lambda e: (e, 0)),
        ],
        out_shape=[
            jax.ShapeDtypeStruct((E, FC), jnp.float32),
            jax.ShapeDtypeStruct((E, FC), jnp.float32),
        ],
    )(edge_attr, edge_nei_len, edge_nei_angle, wfeat, bfeat, wqkv, wkvla, wee)
    return out


CW = 144
CCH = 40
NSUB = 16
NCORE = 2
EPW = E // (NCORE * NSUB)
NCHUNK = EPW // CCH
NPAD = 10240
RPS = NPAD // NSUB
_INV_SQRT_FC = np.float32(1.0 / np.sqrt(FC))


def _sc_conv_body(kv, qn, ee, src3, dst3, out,
                  acc, idx_s, idx_d, kvb, qb, eb, ob, sem):
    c = lax.axis_index("c")
    s = lax.axis_index("s")
    w = c * NSUB + s

    def zrow(r, carry):
        for j in range(CW // 16):
            ob[r, pl.ds(16 * j, 16)] = jnp.zeros((16,), jnp.float32)
        return carry

    lax.fori_loop(0, CCH, zrow, 0)

    def zcopy(t, carry):
        pltpu.sync_copy(ob, acc.at[pl.ds(s * RPS + t * CCH, CCH)])
        return carry

    lax.fori_loop(0, RPS // CCH, zcopy, 0)
    pltpu.sync_copy(src3.at[w], idx_s)
    pltpu.sync_copy(dst3.at[w], idx_d)
    plsc.subcore_barrier()

    base0 = w * EPW

    def chunk(t, carry):
        b = base0 + t * CCH
        d1 = pltpu.async_copy(kv.at[idx_s.at[t]], kvb, sem)
        d2 = pltpu.async_copy(qn.at[idx_d.at[t]], qb, sem)
        d3 = pltpu.async_copy(ee.at[pl.ds(b, CCH)], eb, sem)
        d1.wait()
        d2.wait()
        d3.wait()

        @plsc.parallel_loop(0, CCH)
        def edge(e):
            sacc = jnp.zeros((16,), jnp.float32)
            vstage = []
            for j in range(8):
                sl = pl.ds(16 * j, 16)
                ev = eb[e, sl]
                sacc = sacc + qb[e, sl] * (kvb[e, sl] + ev)
                vstage.append(kvb[e, pl.ds(FC + 16 * j, 16)] + ev)
            sv = jnp.sum(sacc) * _INV_SQRT_FC
            exv = jnp.exp(jnp.broadcast_to(sv, (16,)))
            for j in range(8):
                ob[e, pl.ds(16 * j, 16)] = exv * vstage[j]
            lane = lax.broadcasted_iota(jnp.int32, (16,), 0)
            ob[e, pl.ds(FC, 16)] = jnp.where(lane == 0, exv, 0.0)

        pltpu.sync_copy(ob, acc.at[idx_d.at[t]], add=True)
        return carry

    lax.fori_loop(0, NCHUNK, chunk, 0)
    plsc.subcore_barrier()
    pltpu.sync_copy(acc.at[pl.ds(s * RPS, RPS)],
                    out.at[c, pl.ds(s * RPS, RPS)])


def _sc_conv(kv, qn, ee, src3, dst3):
    mesh = plsc.VectorSubcoreMesh(core_axis_name="c", subcore_axis_name="s",
                                  num_cores=NCORE, num_subcores=NSUB)
    f = pl.kernel(
        _sc_conv_body,
        mesh=mesh,
        compiler_params=pltpu.CompilerParams(use_tc_tiling_on_sc=False,
                                             needs_layout_passes=False),
        out_type=jax.ShapeDtypeStruct((NCORE, NPAD, CW), jnp.float32),
        scratch_types=[
            pltpu.VMEM_SHARED((NPAD, CW), jnp.float32),
            pltpu.VMEM((NCHUNK, CCH), jnp.int32),
            pltpu.VMEM((NCHUNK, CCH), jnp.int32),
            pltpu.VMEM((CCH, 2 * FC), jnp.float32),
            pltpu.VMEM((CCH, FC), jnp.float32),
            pltpu.VMEM((CCH, FC), jnp.float32),
            pltpu.VMEM((CCH, CW), jnp.float32),
            pltpu.SemaphoreType.DMA,
        ],
    )
    return f(kv, qn, ee, src3, dst3)


NB = 2000


def _node0_kernel(x_ref, wa_ref, ba_ref, wk_ref, node_ref, nw_ref):
    node = _dot(x_ref[...], wa_ref[...]) + ba_ref[...]
    node_ref[...] = node
    nw_ref[...] = _dot(node, wk_ref[...])


def _node0(x, W_atom, b_atom, wkvq0):
    return pl.pallas_call(
        _node0_kernel,
        grid=(N // NB,),
        in_specs=[
            pl.BlockSpec((NB, 92), lambda i: (i, 0)),
            pl.BlockSpec((92, FC), lambda i: (0, 0)),
            pl.BlockSpec((1, FC), lambda i: (0, 0)),
            pl.BlockSpec((FC, 3 * FC), lambda i: (0, 0)),
        ],
        out_specs=[
            pl.BlockSpec((NB, FC), lambda i: (i, 0)),
            pl.BlockSpec((NB, 3 * FC), lambda i: (i, 0)),
        ],
        out_shape=[
            jax.ShapeDtypeStruct((N, FC), jnp.float32),
            jax.ShapeDtypeStruct((N, 3 * FC), jnp.float32),
        ],
    )(x, W_atom, b_atom.reshape(1, FC), wkvq0)


def _combine(acc_ref, node_ref):
    p = acc_ref[0] + acc_ref[1]
    inv = 1.0 / (p[:, FC:FC + 1] + 1e-16)
    return node_ref[...] + p[:, :FC] * inv


def _upd_kernel(acc_ref, nodein_ref, wk_ref, node_ref, nw_ref):
    node = _combine(acc_ref, nodein_ref)
    node_ref[...] = node
    nw_ref[...] = _dot(node, wk_ref[...])


def _node_update(acc, node, wkvq1):
    return pl.pallas_call(
        _upd_kernel,
        grid=(N // NB,),
        in_specs=[
            pl.BlockSpec((2, NB, CW), lambda i: (0, i, 0)),
            pl.BlockSpec((NB, FC), lambda i: (i, 0)),
            pl.BlockSpec((FC, 3 * FC), lambda i: (0, 0)),
        ],
        out_specs=[
            pl.BlockSpec((NB, FC), lambda i: (i, 0)),
            pl.BlockSpec((NB, 3 * FC), lambda i: (i, 0)),
        ],
        out_shape=[
            jax.ShapeDtypeStruct((N, FC), jnp.float32),
            jax.ShapeDtypeStruct((N, 3 * FC), jnp.float32),
        ],
    )(acc, node, wkvq1)


def _pool_kernel(acc_ref, nodein_ref, batchf_ref, out_ref):
    node = _combine(acc_ref, nodein_ref)
    gids = lax.broadcasted_iota(jnp.int32, (NG, 1), 0).astype(jnp.float32)
    onehot = jnp.where(batchf_ref[...] == gids, 1.0, 0.0)
    sums = _dot(onehot, node)
    cnt = jnp.sum(onehot, axis=1, keepdims=True)
    out_ref[...] = sums / jnp.maximum(cnt, 1.0)


def _pool(acc, node, batchf):
    return pl.pallas_call(
        _pool_kernel,
        grid=(1,),
        in_specs=[
            pl.BlockSpec((2, N, CW), lambda i: (0, 0, 0)),
            pl.BlockSpec((N, FC), lambda i: (0, 0)),
            pl.BlockSpec((1, N), lambda i: (0, 0)),
        ],
        out_specs=pl.BlockSpec((NG, FC), lambda i: (0, 0)),
        out_shape=jax.ShapeDtypeStruct((NG, FC), jnp.float32),
    )(acc, node, batchf)


def kernel(x, edge_index, edge_attr, edge_nei_len, edge_nei_angle, batch,
           W_atom, b_atom, W_edge, b_edge, W_angle, b_angle,
           Ue_Wq, Ue_Wk_e, Ue_Wk_l, Ue_Wk_a, Ue_Wv_e, Ue_Wv_l, Ue_Wv_a,
           Tc_Wq, Tc_Wk, Tc_Wv, Tc_We):
    wfeat = jnp.zeros((2 * FC, 2 * FC), jnp.float32)
    wfeat = wfeat.at[:FC, :FC].set(W_edge).at[FC:, FC:].set(W_angle)
    bfeat = jnp.concatenate([b_edge, b_angle]).reshape(1, 2 * FC)
    wqkv = jnp.concatenate([Ue_Wq * np.float32(1.0 / np.sqrt(FC)),
                            Ue_Wk_e, Ue_Wv_e], axis=2)
    top = jnp.concatenate([Ue_Wk_l, Ue_Wv_l], axis=2)
    bot = jnp.concatenate([Ue_Wk_a, Ue_Wv_a], axis=2)
    wkvla = jnp.concatenate([top, bot], axis=1)

    ee1, ee2 = _edge_embeddings(edge_attr, edge_nei_len, edge_nei_angle,
                                wfeat, bfeat, wqkv, wkvla, Tc_We)

    src3 = edge_index[0].reshape(NCORE * NSUB, NCHUNK, CCH)
    dst3 = edge_index[1].reshape(NCORE * NSUB, NCHUNK, CCH)
    wkvq = jnp.concatenate([Tc_Wk, Tc_Wv, Tc_Wq], axis=2)
    node, nw = _node0(x, W_atom, b_atom, wkvq[0])
    acc = _sc_conv(nw[:, :2 * FC], nw[:, 2 * FC:], ee1, src3, dst3)
    node, nw = _node_update(acc, node, wkvq[1])
    acc = _sc_conv(nw[:, :2 * FC], nw[:, 2 * FC:], ee2, src3, dst3)
    batchf = batch.astype(jnp.float32).reshape(1, N)
    return _pool(acc, node, batchf)


# speedup vs baseline: 1.0010x; 1.0010x over previous
import jax
import jax.numpy as jnp
import numpy as np
from jax import lax
from jax.experimental import pallas as pl
from jax.experimental.pallas import tpu as pltpu
from jax.experimental.pallas import tpu_sc as plsc

N = 10000
E = 160000
FC = 128
NG = 64
L = 2

EBLK = 2000


def _centers_step(vmin, vmax):
    step = np.float32((vmax - vmin) / (FC - 1))
    return step


def _rbf(x, vmin, vmax):
    step = _centers_step(vmin, vmax)
    gamma = np.float32(1.0) / (step * step)
    c = vmin + step * lax.broadcasted_iota(jnp.int32, (1, FC), 1).astype(
        jnp.float32)
    d = x - c
    return jnp.exp(-gamma * d * d)


def _softplus(x):
    return jnp.log1p(jnp.exp(x))


def _dot(a, b):
    return jnp.dot(a, b, preferred_element_type=jnp.float32)


def _edge_kernel(ea_ref, el_ref, an_ref, wfeat_ref, bfeat_ref, wqkv_ref,
                 wkvla_ref, wee_ref, ee1_ref, ee2_ref):
    ea = ea_ref[...]
    d = jnp.sqrt(jnp.sum(ea * ea, axis=1, keepdims=True))
    x_ef = jnp.float32(-0.75) / d
    f_ef = _rbf(x_ef, -4.0, 4.0)
    ef = _softplus(_dot(f_ef, wfeat_ref[:FC, :FC]) + bfeat_ref[:, :FC])

    nla = []
    for n in range(3):
        f_l = _rbf(el_ref[:, n:n + 1], -4.0, 4.0)
        f_a = _rbf(an_ref[:, n:n + 1], -1.0, 1.0)
        f_cat = jnp.concatenate([f_l, f_a], axis=1)
        nla.append(_softplus(_dot(f_cat, wfeat_ref[...]) + bfeat_ref[...]))

    for i in range(L):
        qkv = _dot(ef, wqkv_ref[i])
        q = qkv[:, :FC]
        kb = qkv[:, FC:2 * FC]
        vb = qkv[:, 2 * FC:]
        s = []
        v = []
        for n in range(3):
            kv = _dot(nla[n], wkvla_ref[i])
            s.append(jnp.sum(q * (kb + kv[:, :FC]), axis=1, keepdims=True))
            v.append(vb + kv[:, FC:])
        s0, s1, s2 = s
        e0 = jnp.exp(s0)
        e1 = jnp.exp(s1)
        e2 = jnp.exp(s2)
        inv_den = 1.0 / (e0 + e1 + e2)
        w0 = e0 * inv_den
        w1 = e1 * inv_den
        w2 = e2 * inv_den
        ef = ef + (w0 * v[0] + w1 * v[1] + w2 * v[2])
        out_ref = ee1_ref if i == 0 else ee2_ref
        out_ref[...] = _dot(ef, wee_ref[i])


def _edge_embeddings(edge_attr, edge_nei_len, edge_nei_angle, wfeat, bfeat,
                     wqkv, wkvla, wee):
    grid = (E // EBLK,)
    eb = pl.BlockSpec((EBLK, 3), lambda e: (e, 0))
    wspec = lambda shape: pl.BlockSpec(shape, lambda e: (0,) * len(shape))
    out = pl.pallas_call(
        _edge_kernel,
        grid=grid,
        in_specs=[
            eb, eb, eb,
            wspec((2 * FC, 2 * FC)),
            wspec((1, 2 * FC)),
            wspec((L, FC, 3 * FC)),
            wspec((L, 2 * FC, 2 * FC)),
            wspec((L, FC, FC)),
        ],
        out_specs=[
            pl.BlockSpec((EBLK, FC), lambda e: (e, 0)),
            pl.BlockSpec((EBLK, FC), lambda e: (e, 0)),
        ],
        out_shape=[
            jax.ShapeDtypeStruct((E, FC), jnp.float32),
            jax.ShapeDtypeStruct((E, FC), jnp.float32),
        ],
    )(edge_attr, edge_nei_len, edge_nei_angle, wfeat, bfeat, wqkv, wkvla, wee)
    return out


CW = 144
CCH = 40
NSUB = 16
NCORE = 2
EPW = E // (NCORE * NSUB)
NCHUNK = EPW // CCH
NPAD = 10240
RPS = NPAD // NSUB
_INV_SQRT_FC = np.float32(1.0 / np.sqrt(FC))


def _sc_conv_body(kv, qn, ee, src3, dst3, out,
                  acc, idx_s, idx_d, kvb, qb, eb, ob, sem):
    c = lax.axis_index("c")
    s = lax.axis_index("s")
    w = c * NSUB + s

    def zrow(r, carry):
        for j in range(CW // 16):
            ob[r, pl.ds(16 * j, 16)] = jnp.zeros((16,), jnp.float32)
        return carry

    lax.fori_loop(0, CCH, zrow, 0)

    def zcopy(t, carry):
        pltpu.sync_copy(ob, acc.at[pl.ds(s * RPS + t * CCH, CCH)])
        return carry

    lax.fori_loop(0, RPS // CCH, zcopy, 0)
    pltpu.sync_copy(src3.at[w], idx_s)
    pltpu.sync_copy(dst3.at[w], idx_d)
    plsc.subcore_barrier()

    base0 = w * EPW

    def chunk(t, carry):
        b = base0 + t * CCH
        d1 = pltpu.async_copy(kv.at[idx_s.at[t]], kvb, sem)
        d2 = pltpu.async_copy(qn.at[idx_d.at[t]], qb, sem)
        d3 = pltpu.async_copy(ee.at[pl.ds(b, CCH)], eb, sem)
        d1.wait()
        d2.wait()
        d3.wait()

        @plsc.parallel_loop(0, CCH)
        def edge(e):
            sacc = jnp.zeros((16,), jnp.float32)
            vstage = []
            for j in range(8):
                sl = pl.ds(16 * j, 16)
                ev = eb[e, sl]
                sacc = sacc + qb[e, sl] * (kvb[e, sl] + ev)
                vstage.append(kvb[e, pl.ds(FC + 16 * j, 16)] + ev)
            sv = jnp.sum(sacc) * _INV_SQRT_FC
            exv = jnp.exp(jnp.broadcast_to(sv, (16,)))
            for j in range(8):
                ob[e, pl.ds(16 * j, 16)] = exv * vstage[j]
            lane = lax.broadcasted_iota(jnp.int32, (16,), 0)
            ob[e, pl.ds(FC, 16)] = jnp.where(lane == 0, exv, 0.0)

        pltpu.sync_copy(ob, acc.at[idx_d.at[t]], add=True)
        return carry

    lax.fori_loop(0, NCHUNK, chunk, 0)
    plsc.subcore_barrier()
    pltpu.sync_copy(acc.at[pl.ds(s * RPS, RPS)],
                    out.at[c, pl.ds(s * RPS, RPS)])


def _sc_conv(kv, qn, ee, src3, dst3):
    mesh = plsc.VectorSubcoreMesh(core_axis_name="c", subcore_axis_name="s",
                                  num_cores=NCORE, num_subcores=NSUB)
    f = pl.kernel(
        _sc_conv_body,
        mesh=mesh,
        compiler_params=pltpu.CompilerParams(use_tc_tiling_on_sc=False,
                                             needs_layout_passes=False),
        out_type=jax.ShapeDtypeStruct((NCORE, NPAD, CW), jnp.float32),
        scratch_types=[
            pltpu.VMEM_SHARED((NPAD, CW), jnp.float32),
            pltpu.VMEM((NCHUNK, CCH), jnp.int32),
            pltpu.VMEM((NCHUNK, CCH), jnp.int32),
            pltpu.VMEM((CCH, 2 * FC), jnp.float32),
            pltpu.VMEM((CCH, FC), jnp.float32),
            pltpu.VMEM((CCH, FC), jnp.float32),
            pltpu.VMEM((CCH, CW), jnp.float32),
            pltpu.SemaphoreType.DMA,
        ],
    )
    return f(kv, qn, ee, src3, dst3)


NB = 2000


def _node0_kernel(x_ref, wa_ref, ba_ref, wk_ref, node_ref, nw_ref):
    node = _dot(x_ref[...], wa_ref[...]) + ba_ref[...]
    node_ref[...] = node
    nw_ref[...] = _dot(node, wk_ref[...])


def _node0(x, W_atom, b_atom, wkvq0):
    return pl.pallas_call(
        _node0_kernel,
        grid=(N // NB,),
        in_specs=[
            pl.BlockSpec((NB, 92), lambda i: (i, 0)),
            pl.BlockSpec((92, FC), lambda i: (0, 0)),
            pl.BlockSpec((1, FC), lambda i: (0, 0)),
            pl.BlockSpec((FC, 3 * FC), lambda i: (0, 0)),
        ],
        out_specs=[
            pl.BlockSpec((NB, FC), lambda i: (i, 0)),
            pl.BlockSpec((NB, 3 * FC), lambda i: (i, 0)),
        ],
        out_shape=[
            jax.ShapeDtypeStruct((N, FC), jnp.float32),
            jax.ShapeDtypeStruct((N, 3 * FC), jnp.float32),
        ],
    )(x, W_atom, b_atom.reshape(1, FC), wkvq0)


def _combine(acc_ref, node_ref):
    p = acc_ref[0] + acc_ref[1]
    inv = 1.0 / (p[:, FC:FC + 1] + 1e-16)
    return node_ref[...] + p[:, :FC] * inv


def _upd_kernel(acc_ref, nodein_ref, wk_ref, node_ref, nw_ref):
    node = _combine(acc_ref, nodein_ref)
    node_ref[...] = node
    nw_ref[...] = _dot(node, wk_ref[...])


def _node_update(acc, node, wkvq1):
    return pl.pallas_call(
        _upd_kernel,
        grid=(N // NB,),
        in_specs=[
            pl.BlockSpec((2, NB, CW), lambda i: (0, i, 0)),
            pl.BlockSpec((NB, FC), lambda i: (i, 0)),
            pl.BlockSpec((FC, 3 * FC), lambda i: (0, 0)),
        ],
        out_specs=[
            pl.BlockSpec((NB, FC), lambda i: (i, 0)),
            pl.BlockSpec((NB, 3 * FC), lambda i: (i, 0)),
        ],
        out_shape=[
            jax.ShapeDtypeStruct((N, FC), jnp.float32),
            jax.ShapeDtypeStruct((N, 3 * FC), jnp.float32),
        ],
    )(acc, node, wkvq1)


def _pool_kernel(acc_ref, nodein_ref, batchf_ref, out_ref):
    node = _combine(acc_ref, nodein_ref)
    gids = lax.broadcasted_iota(jnp.int32, (NG, 1), 0).astype(jnp.float32)
    onehot = jnp.where(batchf_ref[...] == gids, 1.0, 0.0)
    sums = _dot(onehot, node)
    cnt = jnp.sum(onehot, axis=1, keepdims=True)
    out_ref[...] = sums / jnp.maximum(cnt, 1.0)


def _pool(acc, node, batchf):
    return pl.pallas_call(
        _pool_kernel,
        grid=(1,),
        in_specs=[
            pl.BlockSpec((2, N, CW), lambda i: (0, 0, 0)),
            pl.BlockSpec((N, FC), lambda i: (0, 0)),
            pl.BlockSpec((1, N), lambda i: (0, 0)),
        ],
        out_specs=pl.BlockSpec((NG, FC), lambda i: (0, 0)),
        out_shape=jax.ShapeDtypeStruct((NG, FC), jnp.float32),
    )(acc, node, batchf)


def kernel(x, edge_index, edge_attr, edge_nei_len, edge_nei_angle, batch,
           W_atom, b_atom, W_edge, b_edge, W_angle, b_angle,
           Ue_Wq, Ue_Wk_e, Ue_Wk_l, Ue_Wk_a, Ue_Wv_e, Ue_Wv_l, Ue_Wv_a,
           Tc_Wq, Tc_Wk, Tc_Wv, Tc_We):
    wfeat = jnp.zeros((2 * FC, 2 * FC), jnp.float32)
    wfeat = wfeat.at[:FC, :FC].set(W_edge).at[FC:, FC:].set(W_angle)
    bfeat = jnp.concatenate([b_edge, b_angle]).reshape(1, 2 * FC)
    wqkv = jnp.concatenate([Ue_Wq * np.float32(1.0 / np.sqrt(FC)),
                            Ue_Wk_e, Ue_Wv_e], axis=2)
    top = jnp.concatenate([Ue_Wk_l, Ue_Wv_l], axis=2)
    bot = jnp.concatenate([Ue_Wk_a, Ue_Wv_a], axis=2)
    wkvla = jnp.concatenate([top, bot], axis=1)

    ee1, ee2 = _edge_embeddings(edge_attr, edge_nei_len, edge_nei_angle,
                                wfeat, bfeat, wqkv, wkvla, Tc_We)

    src3 = edge_index[0].reshape(NCORE * NSUB, NCHUNK, CCH)
    dst3 = edge_index[1].reshape(NCORE * NSUB, NCHUNK, CCH)
    wkvq = jnp.concatenate([Tc_Wk, Tc_Wv, Tc_Wq], axis=2)
    node, nw = _node0(x, W_atom, b_atom, wkvq[0])
    acc = _sc_conv(nw[:, :2 * FC], nw[:, 2 * FC:], ee1, src3, dst3)
    node, nw = _node_update(acc, node, wkvq[1])
    acc = _sc_conv(nw[:, :2 * FC], nw[:, 2 * FC:], ee2, src3, dst3)
    batchf = batch.astype(jnp.float32).reshape(1, N)
    return _pool(acc, node, batchf)
